# baseline (speedup 1.0000x reference)
import functools
import math

import jax
import jax.numpy as jnp
import numpy as np
from jax import lax
from jax.experimental import pallas as pl
from jax.experimental.pallas import tpu as pltpu
from jax.experimental.pallas import tpu_sc as plsc

_MAX_NORM = 2.0
_W = 64
_GW = 5
_GR = _W * _GW

_S1 = 9.99999981e-01
_S3 = -1.66666497e-01
_S5 = 8.33292673e-03
_S7 = -1.98022542e-04
_S9 = 2.59281518e-06
_HALF_PI = 1.5707963267948966

_ROWS_PER_STEP = 1024


def _sc_gather(table, idx):
    n = idx.shape[0]
    d = table.shape[1]
    info = plsc.get_sparse_core_info()
    nc, ns = info.num_cores, info.num_subcores
    nw = nc * ns
    rpw = n // nw
    assert n % nw == 0 and rpw % _GR == 0
    ngroups = rpw // _GR
    assert ngroups % 2 == 0
    mesh = plsc.VectorSubcoreMesh(core_axis_name="c", subcore_axis_name="s")

    @functools.partial(
        pl.kernel,
        out_type=jax.ShapeDtypeStruct((n, d), table.dtype),
        mesh=mesh,
        compiler_params=pltpu.CompilerParams(use_tc_tiling_on_sc=False),
        scratch_types=[
            pltpu.VMEM((rpw,), jnp.int32),
            pltpu.VMEM((_GR, d), jnp.float32),
            pltpu.VMEM((_GR, d), jnp.float32),
            pltpu.SemaphoreType.DMA,
            pltpu.SemaphoreType.DMA,
            pltpu.SemaphoreType.DMA,
        ],
    )
    def gather_kernel(x_hbm, i_hbm, o_hbm, idx_v, buf0, buf1, gsem, wsem0, wsem1):
        wid = lax.axis_index("s") * nc + lax.axis_index("c")
        base = wid * rpw
        pltpu.sync_copy(i_hbm.at[pl.ds(base, rpw)], idx_v)
        bufs = (buf0, buf1)
        wsems = (wsem0, wsem1)

        def run_group(g, p):
            buf = bufs[p]
            handles = [
                pltpu.async_copy(
                    x_hbm.at[idx_v.at[pl.ds(g * _GR + w * _W, _W)]],
                    buf.at[pl.ds(w * _W, _W)],
                    gsem,
                )
                for w in range(_GW)
            ]
            for h in handles:
                h.wait()
            return pltpu.async_copy(
                buf,
                o_hbm.at[pl.ds(base + g * _GR, _GR)],
                wsems[p],
            )

        def outer(i, carry):
            g0 = i * 2
            writes = [run_group(g0 + p, p) for p in range(2)]
            for h in writes:
                h.wait()
            return carry

        lax.fori_loop(0, ngroups // 2, outer, 0)

    return gather_kernel(table, idx)


def _sin_poly(x):
    x2 = x * x
    return ((((_S9 * x2 + _S7) * x2 + _S5) * x2 + _S3) * x2 + _S1) * x


def _finish_body(g_ref, tv_ref, out_ref):
    x = g_ref[...]
    r = x.shape[0]
    d = x.shape[1]
    rl = r // 128

    def expand(v):
        v3 = v.reshape(rl, 1, 128)
        return jnp.broadcast_to(v3, (rl, 128, 128)).reshape(r, 128)

    sub = lax.broadcasted_iota(jnp.int32, (r, 128), 0) % 128
    lane_r = lax.broadcasted_iota(jnp.int32, (r, 128), 1)
    msel = (sub == lane_r).astype(jnp.float32)
    ones_bc = jnp.ones((128, d), jnp.float32)
    t = jnp.dot(expand(tv_ref[...]) * msel, ones_bc,
                preferred_element_type=jnp.float32)
    x2 = x * x
    ones_n = jnp.ones((d, d), jnp.float32)
    s = jnp.dot(x2, ones_n, preferred_element_type=jnp.float32)
    scale = jnp.where(
        s > _MAX_NORM * _MAX_NORM, _MAX_NORM * lax.rsqrt(s), 1.0
    ) * math.sqrt(d)
    lane = lax.broadcasted_iota(jnp.int32, (1, d), 1)
    jm = lane.astype(jnp.float32)
    coef = jm / (10000.0 ** (2.0 * jm / d))
    phase = t * coef
    arg = jnp.where(lane % 2 == 1, _HALF_PI - phase, phase)
    out_ref[...] = x * scale + _sin_poly(arg)


def _tc_finish(g, tv):
    n, d = g.shape
    r = _ROWS_PER_STEP
    rl = r // 128
    return pl.pallas_call(
        _finish_body,
        grid=(n // r,),
        in_specs=[
            pl.BlockSpec((r, d), lambda i: (i, 0)),
            pl.BlockSpec((rl, 128), lambda i: (i, 0)),
        ],
        out_specs=pl.BlockSpec((r, d), lambda i: (i, 0)),
        out_shape=jax.ShapeDtypeStruct((n, d), jnp.float32),
    )(g, tv)


def kernel(mz_batch, int_batch, table):
    b, l = mz_batch.shape
    d = table.shape[1]
    n = b * l
    idx = mz_batch.reshape(-1).astype(jnp.int32)
    tv = int_batch.reshape(n // 128, 128)
    g = _sc_gather(table, idx)
    out = _tc_finish(g, tv)
    return out.reshape(b, l, d)
